# Initial kernel scaffold; baseline (speedup 1.0000x reference)
import jax
import jax.numpy as jnp
from jax.experimental import pallas as pl


def kernel(sports_ids, age, gender, preferred_gender, gym_days, sport_table, gender_table, gym_table, W1, b1, W2, b2):
    raise NotImplementedError("write your pallas kernel here")


# speedup vs baseline: 28.1206x; 28.1206x over previous
import functools

import jax
import jax.numpy as jnp
from jax import lax
from jax.experimental import pallas as pl
from jax.experimental.pallas import tpu as pltpu
from jax.experimental.pallas import tpu_sc as plsc

_NC = 2
_NS = 16
_NW = _NC * _NS
_FW = 64


def _sc_counts_body(rows, n_ids, idsT_hbm, f_hbm, ids_v, f_v):
  wid = lax.axis_index("s") * _NC + lax.axis_index("c")
  base = wid * rows
  pltpu.sync_copy(idsT_hbm.at[:, pl.ds(base, rows)], ids_v)

  zeros = jnp.zeros((16,), jnp.float32)

  def zero_body(i, carry):
    for u in range(16):
      f_v[pl.ds(i * 256 + u * 16, 16)] = zeros
    return carry

  lax.fori_loop(0, rows * _FW // 256, zero_body, 0)

  ones = jnp.ones((16,), jnp.float32)
  iota_fw = lax.iota(jnp.int32, 16) * _FW

  def group_body(g, carry):
    lane_base = iota_fw + g * (16 * _FW)
    for l in range(n_ids):
      idx = ids_v[l, pl.ds(g * 16, 16)] + lane_base
      plsc.addupdate_scatter(f_v, [idx], ones)
    return carry

  lax.fori_loop(0, rows // 16, group_body, 0)

  pltpu.sync_copy(f_v, f_hbm.at[pl.ds(base * _FW, rows * _FW)])


def _sc_counts(idsT, B):
  rows = B // _NW
  n_ids = idsT.shape[0]
  mesh = plsc.VectorSubcoreMesh(core_axis_name="c", subcore_axis_name="s",
                                num_cores=_NC, num_subcores=_NS)
  return pl.kernel(
      functools.partial(_sc_counts_body, rows, n_ids),
      out_type=jax.ShapeDtypeStruct((B * _FW,), jnp.float32),
      mesh=mesh,
      scratch_types=[
          pltpu.VMEM((n_ids, rows), jnp.int32),
          pltpu.VMEM((rows * _FW,), jnp.float32),
      ],
      compiler_params=pltpu.CompilerParams(needs_layout_passes=False),
  )(idsT)


def _tc_mlp_body(f_ref, age_ref, g_ref, p_ref, st_ref, gt_ref, gyt_ref,
                 w1a_ref, w1gd_ref, w1pf_ref, w1gy_ref, wage_ref, b1_ref,
                 w2_ref, b2_ref, out_ref):
  f32 = jnp.float32
  t_sport = jnp.dot(st_ref[...], w1a_ref[...], preferred_element_type=f32)
  t_gym = jnp.dot(gyt_ref[...], w1gy_ref[...], preferred_element_type=f32)
  m = jnp.concatenate(
      [t_sport * (1.0 / 20.0), t_gym * (1.0 / 7.0),
       jnp.zeros((4, 64), f32)], axis=0)
  h = jnp.dot(f_ref[...], m, preferred_element_type=f32)
  a_gd = jnp.dot(gt_ref[...], w1gd_ref[...], preferred_element_type=f32)
  a_pf = jnp.dot(gt_ref[...], w1pf_ref[...], preferred_element_type=f32)
  h = h + jnp.where(g_ref[...] == 0, a_gd[0:1, :], a_gd[1:2, :])
  h = h + jnp.where(p_ref[...] == 0, a_pf[0:1, :], a_pf[1:2, :])
  age_n = (age_ref[...] - 19.0) * (1.0 / 6.5)
  h = h + age_n * wage_ref[...]
  h = jnp.maximum(h + b1_ref[...], 0.0)
  out_ref[...] = jnp.dot(h, w2_ref[...], preferred_element_type=f32) + b2_ref[...]


def _tc_mlp(F, age, gender, pref, sport_table, gender_table, gym_table,
            w1a, w1gd, w1pf, w1gy, w_age, b1, W2, b2):
  B = F.shape[0]
  blk = 2048
  grid = (B // blk,)
  row_spec = lambda shape: pl.BlockSpec((blk,) + shape[1:],
                                        lambda i: (i,) + (0,) * (len(shape) - 1))
  rep_spec = lambda shape: pl.BlockSpec(shape, lambda i: (0,) * len(shape))
  in_specs = [
      row_spec(F.shape),
      row_spec(age.shape),
      row_spec(gender.shape),
      row_spec(pref.shape),
      rep_spec(sport_table.shape),
      rep_spec(gender_table.shape),
      rep_spec(gym_table.shape),
      rep_spec(w1a.shape),
      rep_spec(w1gd.shape),
      rep_spec(w1pf.shape),
      rep_spec(w1gy.shape),
      rep_spec(w_age.shape),
      rep_spec(b1.shape),
      rep_spec(W2.shape),
      rep_spec(b2.shape),
  ]
  return pl.pallas_call(
      _tc_mlp_body,
      grid=grid,
      in_specs=in_specs,
      out_specs=pl.BlockSpec((blk, 32), lambda i: (i, 0)),
      out_shape=jax.ShapeDtypeStruct((B, 32), jnp.float32),
      compiler_params=pltpu.CompilerParams(
          dimension_semantics=("parallel",)),
  )(F, age, gender, pref, sport_table, gender_table, gym_table,
    w1a, w1gd, w1pf, w1gy, w_age, b1, W2, b2)


def kernel(sports_ids, age, gender, preferred_gender, gym_days,
           sport_table, gender_table, gym_table, W1, b1, W2, b2):
  B = sports_ids.shape[0]
  idsT = jnp.concatenate(
      [sports_ids.astype(jnp.int32).T,
       gym_days.astype(jnp.int32).T + 52], axis=0)
  f_flat = _sc_counts(idsT, B)
  F = f_flat.reshape(B, _FW)
  out = _tc_mlp(
      F, age, gender.astype(jnp.int32), preferred_gender.astype(jnp.int32),
      sport_table, gender_table, gym_table,
      W1[0:10], W1[10:14], W1[14:18], W1[18:22], W1[22:23],
      b1.reshape(1, 64), W2, b2.reshape(1, 32))
  return out
